# Initial kernel scaffold; baseline (speedup 1.0000x reference)
import jax
import jax.numpy as jnp
from jax.experimental import pallas as pl


def kernel(emb, Wx, Wh, b_lstm, W_if, b_if, W_out, b_out, W_fc, b_fc, story_batch, option1_batch, option2_batch):
    raise NotImplementedError("write your pallas kernel here")


# speedup vs baseline: 1.0061x; 1.0061x over previous
import functools

import jax
import jax.numpy as jnp
from jax import lax
from jax.experimental import pallas as pl
from jax.experimental.pallas import tpu as pltpu
from jax.experimental.pallas import tpu_sc as plsc

EMBED_DIM = 61
XDIM = 64
H = 128
N_SLOTS = 64
W_MEM = 64
CLIP = 20.0
B = 256
T_STORY = 50
T_OPT = 20

NW = 32
TOT_IDS = B * (T_STORY + 2 * T_OPT)
B_PER_W = TOT_IDS // NW
CHUNK = 120
NCHUNK = B_PER_W // CHUNK
TPAD = 128


def _sc_gather_body(table_hbm, idx_hbm, out_hbm, idx_v, rows_v, sem):
    wid = lax.axis_index("s") * 2 + lax.axis_index("c")
    pltpu.sync_copy(idx_hbm.at[wid], idx_v)
    for j in range(NCHUNK):
        pltpu.async_copy(
            table_hbm.at[idx_v.at[j]],
            rows_v.at[pl.ds(j * CHUNK, CHUNK)],
            sem,
        ).wait()
    pltpu.sync_copy(rows_v, out_hbm.at[pl.ds(wid * B_PER_W, B_PER_W)])


def _sc_gather(table, idx3):
    mesh = plsc.VectorSubcoreMesh(core_axis_name="c", subcore_axis_name="s")
    run = functools.partial(
        pl.kernel,
        _sc_gather_body,
        mesh=mesh,
        out_type=jax.ShapeDtypeStruct((TOT_IDS, TPAD), jnp.float32),
        scratch_types=[
            pltpu.VMEM((NCHUNK, CHUNK), jnp.int32),
            pltpu.VMEM((B_PER_W, TPAD), jnp.float32),
            pltpu.SemaphoreType.DMA,
        ],
    )()
    return run(table, idx3)


def _dnc_step(inp, h, c, M, r, Wg, bias, Wif, bif):
    gates = jnp.dot(inp, Wg, preferred_element_type=jnp.float32) + bias
    i_g = gates[:, 0:H]
    f_g = gates[:, H:2 * H]
    g_g = gates[:, 2 * H:3 * H]
    o_g = gates[:, 3 * H:4 * H]
    c_new = jax.nn.sigmoid(f_g) * c + jax.nn.sigmoid(i_g) * jnp.tanh(g_g)
    h_new = jax.nn.sigmoid(o_g) * jnp.tanh(c_new)
    h_new = jnp.clip(h_new, -CLIP, CLIP)

    itf = jnp.dot(h_new, Wif, preferred_element_type=jnp.float32) + bif
    wk = itf[:, 0:64]
    er = jax.nn.sigmoid(itf[:, 64:128])
    wv = itf[:, 128:192]
    rk = itf[:, 192:256]
    wb = jnp.logaddexp(itf[:, 256:257], 0.0)
    rb = jnp.logaddexp(itf[:, 257:258], 0.0)

    eps = jnp.float32(1e-6)
    Mnorm = jnp.sqrt(jnp.sum(M * M, axis=-1)) + eps
    wknorm = jnp.sqrt(jnp.sum(wk * wk, axis=-1, keepdims=True)) + eps
    cw = jnp.sum(M * wk[:, None, :], axis=-1) / (Mnorm * wknorm)
    ww = jax.nn.softmax(cw * wb, axis=-1)

    M_new = M * (1.0 - ww[:, :, None] * er[:, None, :]) + ww[:, :, None] * wv[:, None, :]

    Mnorm2 = jnp.sqrt(jnp.sum(M_new * M_new, axis=-1)) + eps
    rknorm = jnp.sqrt(jnp.sum(rk * rk, axis=-1, keepdims=True)) + eps
    cr = jnp.sum(M_new * rk[:, None, :], axis=-1) / (Mnorm2 * rknorm)
    wr = jax.nn.softmax(cr * rb, axis=-1)
    r_new = jnp.sum(wr[:, :, None] * M_new, axis=1)
    return h_new, c_new, M_new, r_new, wr, ww


GS = 2
BS = B // GS


def _story_body(x_ref, Wg_ref, b_ref, Wif_ref, bif_ref,
                wr_ref, ww_ref, ho_ref, co_ref, Mo_ref, ro_ref):
    Wg = Wg_ref[...]
    Wif = Wif_ref[...]
    bias = b_ref[...]
    bif = bif_ref[...]

    def step(t, carry):
        h, c, M, r = carry
        x = x_ref[t]
        inp = jnp.concatenate([x, r, h], axis=-1)
        h, c, M, r, wr, ww = _dnc_step(inp, h, c, M, r, Wg, bias, Wif, bif)
        wr_ref[t] = wr
        ww_ref[t] = ww
        return (h, c, M, r)

    init = (jnp.zeros((BS, H), jnp.float32), jnp.zeros((BS, H), jnp.float32),
            jnp.zeros((BS, N_SLOTS, W_MEM), jnp.float32), jnp.zeros((BS, W_MEM), jnp.float32))
    h, c, M, r = lax.fori_loop(0, T_STORY, step, init)
    ho_ref[...] = h
    co_ref[...] = c
    Mo_ref[...] = M
    ro_ref[...] = r


def _run_story(sx, Wg, b_story, Wif, bif):
    out_shape = [
        jax.ShapeDtypeStruct((T_STORY, B, N_SLOTS), jnp.float32),
        jax.ShapeDtypeStruct((T_STORY, B, N_SLOTS), jnp.float32),
        jax.ShapeDtypeStruct((B, H), jnp.float32),
        jax.ShapeDtypeStruct((B, H), jnp.float32),
        jax.ShapeDtypeStruct((B, N_SLOTS, W_MEM), jnp.float32),
        jax.ShapeDtypeStruct((B, W_MEM), jnp.float32),
    ]
    in_specs = [
        pl.BlockSpec((T_STORY, BS, XDIM), lambda i: (0, i, 0)),
        pl.BlockSpec((2 * H, 4 * H), lambda i: (0, 0)),
        pl.BlockSpec((1, 4 * H), lambda i: (0, 0)),
        pl.BlockSpec((H, 384), lambda i: (0, 0)),
        pl.BlockSpec((1, 384), lambda i: (0, 0)),
    ]
    out_specs = [
        pl.BlockSpec((T_STORY, BS, N_SLOTS), lambda i: (0, i, 0)),
        pl.BlockSpec((T_STORY, BS, N_SLOTS), lambda i: (0, i, 0)),
        pl.BlockSpec((BS, H), lambda i: (i, 0)),
        pl.BlockSpec((BS, H), lambda i: (i, 0)),
        pl.BlockSpec((BS, N_SLOTS, W_MEM), lambda i: (i, 0, 0)),
        pl.BlockSpec((BS, W_MEM), lambda i: (i, 0)),
    ]
    return pl.pallas_call(
        _story_body, out_shape=out_shape, grid=(GS,),
        in_specs=in_specs, out_specs=out_specs,
        compiler_params=pltpu.CompilerParams(vmem_limit_bytes=100 * 1024 * 1024),
    )(sx, Wg, b_story, Wif, bif)


GO = 4
BO = 2 * B // GO


def _opt_body(x_ref, Wg_ref, b2_ref, bpl_ref, Wif_ref, bif_ref,
              Wout_ref, bout_ref, Wfc_ref, bfc_ref,
              h_ref, c_ref, M_ref, r_ref,
              out_ref, wr_ref, ww_ref, a_scr):
    Wg = Wg_ref[...]
    Wif = Wif_ref[...]
    bif = bif_ref[...]
    bias = b2_ref[0]

    def step(t, carry):
        h, c, M, r = carry
        x = x_ref[t]
        inp = jnp.concatenate([x, r, h], axis=-1)
        h, c, M, r, wr, ww = _dnc_step(inp, h, c, M, r, Wg, bias, Wif, bif)
        wr_ref[t] = wr
        ww_ref[t] = ww
        return (h, c, M, r)

    state0 = (h_ref[...], c_ref[...], M_ref[...], r_ref[...])
    h, c, M, r = lax.fori_loop(0, T_OPT, step, state0)

    inp = jnp.concatenate([jnp.zeros((BO, XDIM), jnp.float32), r, h], axis=-1)
    h, c, M, r, wr, ww = _dnc_step(inp, h, c, M, r, Wg, bpl_ref[...], Wif, bif)
    wr_ref[T_OPT] = wr
    ww_ref[T_OPT] = ww

    a = jnp.dot(jnp.concatenate([h, r], axis=-1), Wout_ref[...],
                preferred_element_type=jnp.float32) + bout_ref[...]
    i = pl.program_id(0)

    @pl.when(i % 2 == 0)
    def _():
        a_scr[...] = a

    @pl.when(i % 2 == 1)
    def _():
        combined = jnp.concatenate([a_scr[...], a], axis=-1)
        out_ref[...] = jnp.dot(combined, Wfc_ref[...],
                               preferred_element_type=jnp.float32) + bfc_ref[...]


def _run_options(ox, Wg, b2, b_plain, Wif, bif, Wout, bout, Wfc, bfc,
                 h, c, M, r):
    out_shape = [
        jax.ShapeDtypeStruct((B, 2), jnp.float32),
        jax.ShapeDtypeStruct((T_OPT + 1, 2 * B, N_SLOTS), jnp.float32),
        jax.ShapeDtypeStruct((T_OPT + 1, 2 * B, N_SLOTS), jnp.float32),
    ]
    in_specs = [
        pl.BlockSpec((T_OPT, BO, XDIM), lambda i: (0, i, 0)),
        pl.BlockSpec((2 * H, 4 * H), lambda i: (0, 0)),
        pl.BlockSpec((1, 1, 4 * H), lambda i: (i % 2, 0, 0)),
        pl.BlockSpec((1, 4 * H), lambda i: (0, 0)),
        pl.BlockSpec((H, 384), lambda i: (0, 0)),
        pl.BlockSpec((1, 384), lambda i: (0, 0)),
        pl.BlockSpec((H + W_MEM, W_MEM), lambda i: (0, 0)),
        pl.BlockSpec((1, W_MEM), lambda i: (0, 0)),
        pl.BlockSpec((H, 2), lambda i: (0, 0)),
        pl.BlockSpec((1, 2), lambda i: (0, 0)),
        pl.BlockSpec((BO, H), lambda i: (i // 2, 0)),
        pl.BlockSpec((BO, H), lambda i: (i // 2, 0)),
        pl.BlockSpec((BO, N_SLOTS, W_MEM), lambda i: (i // 2, 0, 0)),
        pl.BlockSpec((BO, W_MEM), lambda i: (i // 2, 0)),
    ]
    out_specs = [
        pl.BlockSpec((BO, 2), lambda i: (i // 2, 0)),
        pl.BlockSpec((T_OPT + 1, BO, N_SLOTS), lambda i: (0, i, 0)),
        pl.BlockSpec((T_OPT + 1, BO, N_SLOTS), lambda i: (0, i, 0)),
    ]
    return pl.pallas_call(
        _opt_body, out_shape=out_shape, grid=(GO,),
        in_specs=in_specs, out_specs=out_specs,
        scratch_shapes=[pltpu.VMEM((BO, W_MEM), jnp.float32)],
        compiler_params=pltpu.CompilerParams(vmem_limit_bytes=100 * 1024 * 1024),
    )(ox, Wg, b2, b_plain, Wif, bif, Wout, bout, Wfc, bfc,
      h, c, M, r)


def kernel(emb, Wx, Wh, b_lstm, W_if, b_if, W_out, b_out, W_fc, b_fc,
           story_batch, option1_batch, option2_batch):
    f32 = jnp.float32
    table = jnp.pad(emb, ((0, 0), (0, TPAD - EMBED_DIM)))
    ids = jnp.concatenate([
        story_batch.reshape(-1), option1_batch.reshape(-1), option2_batch.reshape(-1)
    ]).astype(jnp.int32).reshape(NW, NCHUNK, CHUNK)

    g = _sc_gather(table, ids)[:, 0:XDIM]

    n_s = B * T_STORY
    n_o = B * T_OPT
    sx = g[0:n_s].reshape(B, T_STORY, XDIM).transpose(1, 0, 2)
    o1x = g[n_s:n_s + n_o].reshape(B, T_OPT, XDIM)
    o2x = g[n_s + n_o:].reshape(B, T_OPT, XDIM)
    ox = jnp.concatenate([o1x[0:BO], o2x[0:BO], o1x[BO:B], o2x[BO:B]],
                         axis=0).transpose(1, 0, 2)

    Wg = jnp.concatenate([Wx, Wh], axis=0)
    b_story = (b_lstm + Wx[61])[None].astype(f32)
    b_o1 = (b_lstm + Wx[62])[None].astype(f32)
    b_o2 = (b_lstm + Wx[63])[None].astype(f32)
    b_plain = b_lstm[None].astype(f32)
    Wif = jnp.concatenate([
        W_if[:, 0:64], W_if[:, 65:129], W_if[:, 129:193], W_if[:, 193:257],
        W_if[:, 64:65], W_if[:, 257:258], jnp.zeros((H, 126), f32)
    ], axis=1)
    bif = jnp.concatenate([
        b_if[0:64], b_if[65:129], b_if[129:193], b_if[193:257],
        b_if[64:65], b_if[257:258], jnp.zeros((126,), f32)
    ])[None]

    wr_s, ww_s, h, c, M, r = _run_story(sx, Wg, b_story, Wif, bif)
    b2 = jnp.concatenate([b_o1, b_o2], axis=0)[:, None, :]
    logits, wr_o, ww_o = _run_options(
        ox, Wg, b2, b_plain, Wif, bif,
        W_out, b_out[None], W_fc, b_fc[None], h, c, M, r)

    hr_s = wr_s.transpose(1, 0, 2)
    hw_s = ww_s.transpose(1, 0, 2)
    wr_o = wr_o.transpose(1, 0, 2)
    ww_o = ww_o.transpose(1, 0, 2)
    hr1 = jnp.concatenate([wr_o[0:BO], wr_o[2 * BO:3 * BO]], axis=0)
    hr2 = jnp.concatenate([wr_o[BO:2 * BO], wr_o[3 * BO:4 * BO]], axis=0)
    hw1 = jnp.concatenate([ww_o[0:BO], ww_o[2 * BO:3 * BO]], axis=0)
    hw2 = jnp.concatenate([ww_o[BO:2 * BO], ww_o[3 * BO:4 * BO]], axis=0)
    return (logits, (hr_s, hr1, hr2), (hw_s, hw1, hw2))


# speedup vs baseline: 7.3314x; 7.2866x over previous
import functools

import jax
import jax.numpy as jnp
from jax import lax
from jax.experimental import pallas as pl
from jax.experimental.pallas import tpu as pltpu
from jax.experimental.pallas import tpu_sc as plsc

EMBED_DIM = 61
XDIM = 64
H = 128
N_SLOTS = 64
W_MEM = 64
CLIP = 20.0
B = 256
T_STORY = 50
T_OPT = 20

NW = 32
TOT_IDS = B * (T_STORY + 2 * T_OPT)
B_PER_W = TOT_IDS // NW
CHUNK = 120
NCHUNK = B_PER_W // CHUNK
TPAD = 128


def _sc_gather_body(table_hbm, idx_hbm, out_hbm, idx_v, rows_v, sem):
    wid = lax.axis_index("s") * 2 + lax.axis_index("c")
    pltpu.sync_copy(idx_hbm.at[wid], idx_v)
    for j in range(NCHUNK):
        pltpu.async_copy(
            table_hbm.at[idx_v.at[j]],
            rows_v.at[pl.ds(j * CHUNK, CHUNK)],
            sem,
        ).wait()
    pltpu.sync_copy(rows_v, out_hbm.at[pl.ds(wid * B_PER_W, B_PER_W)])


def _sc_gather(table, idx3):
    mesh = plsc.VectorSubcoreMesh(core_axis_name="c", subcore_axis_name="s")
    run = functools.partial(
        pl.kernel,
        _sc_gather_body,
        mesh=mesh,
        out_type=jax.ShapeDtypeStruct((TOT_IDS, TPAD), jnp.float32),
        scratch_types=[
            pltpu.VMEM((NCHUNK, CHUNK), jnp.int32),
            pltpu.VMEM((B_PER_W, TPAD), jnp.float32),
            pltpu.SemaphoreType.DMA,
        ],
    )()
    return run(table, idx3)


def _softmax0(x):
    e = jnp.exp(x - jnp.max(x, axis=0, keepdims=True))
    return e / jnp.sum(e, axis=0, keepdims=True)


def _dnc_step(inp, h, c, MT, r, Mn, Wg, bias, Wif, bif):
    gates = jnp.dot(inp, Wg, preferred_element_type=jnp.float32) + bias
    i_g = gates[:, 0:H]
    f_g = gates[:, H:2 * H]
    g_g = gates[:, 2 * H:3 * H]
    o_g = gates[:, 3 * H:4 * H]
    c_new = jax.nn.sigmoid(f_g) * c + jax.nn.sigmoid(i_g) * jnp.tanh(g_g)
    h_new = jax.nn.sigmoid(o_g) * jnp.tanh(c_new)
    h_new = jnp.clip(h_new, -CLIP, CLIP)

    itf = jnp.dot(h_new, Wif, preferred_element_type=jnp.float32) + bif
    kT = jnp.transpose(itf[:, 0:256])
    wkT = kT[0:64]
    erT = jax.nn.sigmoid(kT[64:128])
    wvT = kT[128:192]
    rkT = kT[192:256]
    bT = jnp.transpose(itf[:, 256:258])
    wbT = jnp.logaddexp(bT[0:1], 0.0)
    rbT = jnp.logaddexp(bT[1:2], 0.0)

    eps = jnp.float32(1e-6)
    wknorm = jnp.sqrt(jnp.sum(wkT * wkT, axis=0, keepdims=True)) + eps
    cw = jnp.sum(MT * wkT[None, :, :], axis=1) / ((Mn + eps) * wknorm)
    wwT = _softmax0(cw * wbT)

    MT_new = MT * (1.0 - wwT[:, None, :] * erT[None, :, :]) + wwT[:, None, :] * wvT[None, :, :]

    Mn2 = jnp.sqrt(jnp.sum(MT_new * MT_new, axis=1))
    rknorm = jnp.sqrt(jnp.sum(rkT * rkT, axis=0, keepdims=True)) + eps
    cr = jnp.sum(MT_new * rkT[None, :, :], axis=1) / ((Mn2 + eps) * rknorm)
    wrT = _softmax0(cr * rbT)
    rT_new = jnp.sum(wrT[:, None, :] * MT_new, axis=0)
    r_new = jnp.transpose(rT_new)
    return h_new, c_new, MT_new, r_new, Mn2, wrT, wwT


GS = 2
BS = B // GS


def _story_body(x_ref, Wg_ref, b_ref, Wif_ref, bif_ref,
                wr_ref, ww_ref, ho_ref, co_ref, Mo_ref, ro_ref, Mno_ref):
    Wg = Wg_ref[...]
    Wif = Wif_ref[...]
    bias = b_ref[...]
    bif = bif_ref[...]

    def step(t, carry):
        h, c, MT, r, Mn = carry
        x = x_ref[t]
        inp = jnp.concatenate([x, r, h], axis=-1)
        h, c, MT, r, Mn, wrT, wwT = _dnc_step(inp, h, c, MT, r, Mn, Wg, bias, Wif, bif)
        wr_ref[t] = wrT
        ww_ref[t] = wwT
        return (h, c, MT, r, Mn)

    init = (jnp.zeros((BS, H), jnp.float32), jnp.zeros((BS, H), jnp.float32),
            jnp.zeros((N_SLOTS, W_MEM, BS), jnp.float32), jnp.zeros((BS, W_MEM), jnp.float32),
            jnp.zeros((N_SLOTS, BS), jnp.float32))
    h, c, MT, r, Mn = lax.fori_loop(0, T_STORY, step, init)
    ho_ref[...] = h
    co_ref[...] = c
    Mo_ref[...] = MT
    ro_ref[...] = r
    Mno_ref[...] = Mn


def _run_story(sx, Wg, b_story, Wif, bif):
    out_shape = [
        jax.ShapeDtypeStruct((T_STORY, N_SLOTS, B), jnp.float32),
        jax.ShapeDtypeStruct((T_STORY, N_SLOTS, B), jnp.float32),
        jax.ShapeDtypeStruct((B, H), jnp.float32),
        jax.ShapeDtypeStruct((B, H), jnp.float32),
        jax.ShapeDtypeStruct((N_SLOTS, W_MEM, B), jnp.float32),
        jax.ShapeDtypeStruct((B, W_MEM), jnp.float32),
        jax.ShapeDtypeStruct((N_SLOTS, B), jnp.float32),
    ]
    in_specs = [
        pl.BlockSpec((T_STORY, BS, XDIM), lambda i: (0, i, 0)),
        pl.BlockSpec((2 * H, 4 * H), lambda i: (0, 0)),
        pl.BlockSpec((1, 4 * H), lambda i: (0, 0)),
        pl.BlockSpec((H, 384), lambda i: (0, 0)),
        pl.BlockSpec((1, 384), lambda i: (0, 0)),
    ]
    out_specs = [
        pl.BlockSpec((T_STORY, N_SLOTS, BS), lambda i: (0, 0, i)),
        pl.BlockSpec((T_STORY, N_SLOTS, BS), lambda i: (0, 0, i)),
        pl.BlockSpec((BS, H), lambda i: (i, 0)),
        pl.BlockSpec((BS, H), lambda i: (i, 0)),
        pl.BlockSpec((N_SLOTS, W_MEM, BS), lambda i: (0, 0, i)),
        pl.BlockSpec((BS, W_MEM), lambda i: (i, 0)),
        pl.BlockSpec((N_SLOTS, BS), lambda i: (0, i)),
    ]
    return pl.pallas_call(
        _story_body, out_shape=out_shape, grid=(GS,),
        in_specs=in_specs, out_specs=out_specs,
        compiler_params=pltpu.CompilerParams(vmem_limit_bytes=100 * 1024 * 1024),
    )(sx, Wg, b_story, Wif, bif)


GO = 4
BO = 2 * B // GO


def _opt_body(x_ref, Wg_ref, b2_ref, bpl_ref, Wif_ref, bif_ref,
              Wout_ref, bout_ref, Wfc_ref, bfc_ref,
              h_ref, c_ref, M_ref, r_ref, Mn_ref,
              out_ref, wr_ref, ww_ref, a_scr):
    Wg = Wg_ref[...]
    Wif = Wif_ref[...]
    bif = bif_ref[...]
    bias = b2_ref[0]

    def step(t, carry):
        h, c, MT, r, Mn = carry
        x = x_ref[t]
        inp = jnp.concatenate([x, r, h], axis=-1)
        h, c, MT, r, Mn, wrT, wwT = _dnc_step(inp, h, c, MT, r, Mn, Wg, bias, Wif, bif)
        wr_ref[t] = wrT
        ww_ref[t] = wwT
        return (h, c, MT, r, Mn)

    state0 = (h_ref[...], c_ref[...], M_ref[...], r_ref[...], Mn_ref[...])
    h, c, MT, r, Mn = lax.fori_loop(0, T_OPT, step, state0)

    inp = jnp.concatenate([jnp.zeros((BO, XDIM), jnp.float32), r, h], axis=-1)
    h, c, MT, r, Mn, wrT, wwT = _dnc_step(inp, h, c, MT, r, Mn, Wg, bpl_ref[...], Wif, bif)
    wr_ref[T_OPT] = wrT
    ww_ref[T_OPT] = wwT

    a = jnp.dot(jnp.concatenate([h, r], axis=-1), Wout_ref[...],
                preferred_element_type=jnp.float32) + bout_ref[...]
    i = pl.program_id(0)

    @pl.when(i % 2 == 0)
    def _():
        a_scr[...] = a

    @pl.when(i % 2 == 1)
    def _():
        combined = jnp.concatenate([a_scr[...], a], axis=-1)
        out_ref[...] = jnp.dot(combined, Wfc_ref[...],
                               preferred_element_type=jnp.float32) + bfc_ref[...]


def _run_options(ox, Wg, b2, b_plain, Wif, bif, Wout, bout, Wfc, bfc,
                 h, c, M, r, Mn):
    out_shape = [
        jax.ShapeDtypeStruct((B, 2), jnp.float32),
        jax.ShapeDtypeStruct((T_OPT + 1, N_SLOTS, 2 * B), jnp.float32),
        jax.ShapeDtypeStruct((T_OPT + 1, N_SLOTS, 2 * B), jnp.float32),
    ]
    in_specs = [
        pl.BlockSpec((T_OPT, BO, XDIM), lambda i: (0, i, 0)),
        pl.BlockSpec((2 * H, 4 * H), lambda i: (0, 0)),
        pl.BlockSpec((1, 1, 4 * H), lambda i: (i % 2, 0, 0)),
        pl.BlockSpec((1, 4 * H), lambda i: (0, 0)),
        pl.BlockSpec((H, 384), lambda i: (0, 0)),
        pl.BlockSpec((1, 384), lambda i: (0, 0)),
        pl.BlockSpec((H + W_MEM, W_MEM), lambda i: (0, 0)),
        pl.BlockSpec((1, W_MEM), lambda i: (0, 0)),
        pl.BlockSpec((H, 2), lambda i: (0, 0)),
        pl.BlockSpec((1, 2), lambda i: (0, 0)),
        pl.BlockSpec((BO, H), lambda i: (i // 2, 0)),
        pl.BlockSpec((BO, H), lambda i: (i // 2, 0)),
        pl.BlockSpec((N_SLOTS, W_MEM, BO), lambda i: (0, 0, i // 2)),
        pl.BlockSpec((BO, W_MEM), lambda i: (i // 2, 0)),
        pl.BlockSpec((N_SLOTS, BO), lambda i: (0, i // 2)),
    ]
    out_specs = [
        pl.BlockSpec((BO, 2), lambda i: (i // 2, 0)),
        pl.BlockSpec((T_OPT + 1, N_SLOTS, BO), lambda i: (0, 0, i)),
        pl.BlockSpec((T_OPT + 1, N_SLOTS, BO), lambda i: (0, 0, i)),
    ]
    return pl.pallas_call(
        _opt_body, out_shape=out_shape, grid=(GO,),
        in_specs=in_specs, out_specs=out_specs,
        scratch_shapes=[pltpu.VMEM((BO, W_MEM), jnp.float32)],
        compiler_params=pltpu.CompilerParams(vmem_limit_bytes=100 * 1024 * 1024),
    )(ox, Wg, b2, b_plain, Wif, bif, Wout, bout, Wfc, bfc,
      h, c, M, r, Mn)


def kernel(emb, Wx, Wh, b_lstm, W_if, b_if, W_out, b_out, W_fc, b_fc,
           story_batch, option1_batch, option2_batch):
    f32 = jnp.float32
    table = jnp.pad(emb, ((0, 0), (0, TPAD - EMBED_DIM)))
    ids = jnp.concatenate([
        story_batch.reshape(-1), option1_batch.reshape(-1), option2_batch.reshape(-1)
    ]).astype(jnp.int32).reshape(NW, NCHUNK, CHUNK)

    g = _sc_gather(table, ids)[:, 0:XDIM]

    n_s = B * T_STORY
    n_o = B * T_OPT
    sx = g[0:n_s].reshape(B, T_STORY, XDIM).transpose(1, 0, 2)
    o1x = g[n_s:n_s + n_o].reshape(B, T_OPT, XDIM)
    o2x = g[n_s + n_o:].reshape(B, T_OPT, XDIM)
    ox = jnp.concatenate([o1x[0:BO], o2x[0:BO], o1x[BO:B], o2x[BO:B]],
                         axis=0).transpose(1, 0, 2)

    Wg = jnp.concatenate([Wx, Wh], axis=0)
    b_story = (b_lstm + Wx[61])[None].astype(f32)
    b_o1 = (b_lstm + Wx[62])[None].astype(f32)
    b_o2 = (b_lstm + Wx[63])[None].astype(f32)
    b_plain = b_lstm[None].astype(f32)
    Wif = jnp.concatenate([
        W_if[:, 0:64], W_if[:, 65:129], W_if[:, 129:193], W_if[:, 193:257],
        W_if[:, 64:65], W_if[:, 257:258], jnp.zeros((H, 126), f32)
    ], axis=1)
    bif = jnp.concatenate([
        b_if[0:64], b_if[65:129], b_if[129:193], b_if[193:257],
        b_if[64:65], b_if[257:258], jnp.zeros((126,), f32)
    ])[None]

    wr_s, ww_s, h, c, M, r, Mn = _run_story(sx, Wg, b_story, Wif, bif)
    b2 = jnp.concatenate([b_o1, b_o2], axis=0)[:, None, :]
    logits, wr_o, ww_o = _run_options(
        ox, Wg, b2, b_plain, Wif, bif,
        W_out, b_out[None], W_fc, b_fc[None], h, c, M, r, Mn)

    hr_s = wr_s.transpose(2, 0, 1)
    hw_s = ww_s.transpose(2, 0, 1)
    wr_o = wr_o.transpose(2, 0, 1)
    ww_o = ww_o.transpose(2, 0, 1)
    hr1 = jnp.concatenate([wr_o[0:BO], wr_o[2 * BO:3 * BO]], axis=0)
    hr2 = jnp.concatenate([wr_o[BO:2 * BO], wr_o[3 * BO:4 * BO]], axis=0)
    hw1 = jnp.concatenate([ww_o[0:BO], ww_o[2 * BO:3 * BO]], axis=0)
    hw2 = jnp.concatenate([ww_o[BO:2 * BO], ww_o[3 * BO:4 * BO]], axis=0)
    return (logits, (hr_s, hr1, hr2), (hw_s, hw1, hw2))


# speedup vs baseline: 8.0479x; 1.0977x over previous
import functools

import jax
import jax.numpy as jnp
from jax import lax
from jax.experimental import pallas as pl
from jax.experimental.pallas import tpu as pltpu
from jax.experimental.pallas import tpu_sc as plsc

EMBED_DIM = 61
XDIM = 64
H = 128
N_SLOTS = 64
W_MEM = 64
CLIP = 20.0
B = 256
T_STORY = 50
T_OPT = 20

NW = 32
TOT_IDS = B * (T_STORY + 2 * T_OPT)
B_PER_W = TOT_IDS // NW
CHUNK = 120
NCHUNK = B_PER_W // CHUNK
TPAD = 128


def _sc_gather_body(table_hbm, idx_hbm, out_hbm, idx_v, rows_v, sem):
    wid = lax.axis_index("s") * 2 + lax.axis_index("c")
    pltpu.sync_copy(idx_hbm.at[wid], idx_v)
    for j in range(NCHUNK):
        pltpu.async_copy(
            table_hbm.at[idx_v.at[j]],
            rows_v.at[pl.ds(j * CHUNK, CHUNK)],
            sem,
        ).wait()
    pltpu.sync_copy(rows_v, out_hbm.at[pl.ds(wid * B_PER_W, B_PER_W)])


def _sc_gather(table, idx3):
    mesh = plsc.VectorSubcoreMesh(core_axis_name="c", subcore_axis_name="s")
    run = functools.partial(
        pl.kernel,
        _sc_gather_body,
        mesh=mesh,
        out_type=jax.ShapeDtypeStruct((TOT_IDS, TPAD), jnp.float32),
        scratch_types=[
            pltpu.VMEM((NCHUNK, CHUNK), jnp.int32),
            pltpu.VMEM((B_PER_W, TPAD), jnp.float32),
            pltpu.SemaphoreType.DMA,
        ],
    )()
    return run(table, idx3)


def _softmax0(x):
    e = jnp.exp(x - jnp.max(x, axis=0, keepdims=True))
    return e / jnp.sum(e, axis=0, keepdims=True)


def _dnc_step(inp, h, c, MT, r, Mn, Wg, bias, Wif, bif):
    gates = jnp.dot(inp, Wg, preferred_element_type=jnp.float32) + bias
    i_g = gates[:, 0:H]
    f_g = gates[:, H:2 * H]
    g_g = gates[:, 2 * H:3 * H]
    o_g = gates[:, 3 * H:4 * H]
    c_new = jax.nn.sigmoid(f_g) * c + jax.nn.sigmoid(i_g) * jnp.tanh(g_g)
    h_new = jax.nn.sigmoid(o_g) * jnp.tanh(c_new)
    h_new = jnp.clip(h_new, -CLIP, CLIP)

    itf = jnp.dot(h_new, Wif, preferred_element_type=jnp.float32) + bif
    kT = jnp.transpose(itf[:, 0:256])
    wkT = kT[0:64]
    erT = jax.nn.sigmoid(kT[64:128])
    wvT = kT[128:192]
    rkT = kT[192:256]
    bT = jnp.transpose(itf[:, 256:258])
    wbT = jnp.logaddexp(bT[0:1], 0.0)
    rbT = jnp.logaddexp(bT[1:2], 0.0)

    eps = jnp.float32(1e-6)
    wknorm = jnp.sqrt(jnp.sum(wkT * wkT, axis=0, keepdims=True)) + eps
    cw = jnp.sum(MT * wkT[None, :, :], axis=1) / ((Mn + eps) * wknorm)
    wwT = _softmax0(cw * wbT)

    MT_new = MT + wwT[:, None, :] * (wvT[None, :, :] - MT * erT[None, :, :])

    Mn2 = jnp.sqrt(jnp.sum(MT_new * MT_new, axis=1))
    rknorm = jnp.sqrt(jnp.sum(rkT * rkT, axis=0, keepdims=True)) + eps
    cr = jnp.sum(MT_new * rkT[None, :, :], axis=1) / ((Mn2 + eps) * rknorm)
    wrT = _softmax0(cr * rbT)
    rT_new = jnp.sum(wrT[:, None, :] * MT_new, axis=0)
    r_new = jnp.transpose(rT_new)
    return h_new, c_new, MT_new, r_new, Mn2, wrT, wwT


GS = 2
BS = B // GS


def _story_body(x_ref, Wg_ref, b_ref, Wif_ref, bif_ref,
                wr_ref, ww_ref, ho_ref, co_ref, Mo_ref, ro_ref, Mno_ref):
    Wg = Wg_ref[...]
    Wif = Wif_ref[...]
    bias = b_ref[...]
    bif = bif_ref[...]

    def step(t, carry):
        h, c, MT, r, Mn = carry
        x = x_ref[t]
        inp = jnp.concatenate([x, r, h], axis=-1)
        h, c, MT, r, Mn, wrT, wwT = _dnc_step(inp, h, c, MT, r, Mn, Wg, bias, Wif, bif)
        wr_ref[t] = wrT
        ww_ref[t] = wwT
        return (h, c, MT, r, Mn)

    init = (jnp.zeros((BS, H), jnp.float32), jnp.zeros((BS, H), jnp.float32),
            jnp.zeros((N_SLOTS, W_MEM, BS), jnp.float32), jnp.zeros((BS, W_MEM), jnp.float32),
            jnp.zeros((N_SLOTS, BS), jnp.float32))
    h, c, MT, r, Mn = lax.fori_loop(0, T_STORY, step, init)
    ho_ref[...] = h
    co_ref[...] = c
    Mo_ref[...] = MT
    ro_ref[...] = r
    Mno_ref[...] = Mn


def _run_story(sx, Wg, b_story, Wif, bif):
    out_shape = [
        jax.ShapeDtypeStruct((T_STORY, N_SLOTS, B), jnp.float32),
        jax.ShapeDtypeStruct((T_STORY, N_SLOTS, B), jnp.float32),
        jax.ShapeDtypeStruct((B, H), jnp.float32),
        jax.ShapeDtypeStruct((B, H), jnp.float32),
        jax.ShapeDtypeStruct((N_SLOTS, W_MEM, B), jnp.float32),
        jax.ShapeDtypeStruct((B, W_MEM), jnp.float32),
        jax.ShapeDtypeStruct((N_SLOTS, B), jnp.float32),
    ]
    in_specs = [
        pl.BlockSpec((T_STORY, BS, XDIM), lambda i: (0, i, 0)),
        pl.BlockSpec((2 * H, 4 * H), lambda i: (0, 0)),
        pl.BlockSpec((1, 4 * H), lambda i: (0, 0)),
        pl.BlockSpec((H, 384), lambda i: (0, 0)),
        pl.BlockSpec((1, 384), lambda i: (0, 0)),
    ]
    out_specs = [
        pl.BlockSpec((T_STORY, N_SLOTS, BS), lambda i: (0, 0, i)),
        pl.BlockSpec((T_STORY, N_SLOTS, BS), lambda i: (0, 0, i)),
        pl.BlockSpec((BS, H), lambda i: (i, 0)),
        pl.BlockSpec((BS, H), lambda i: (i, 0)),
        pl.BlockSpec((N_SLOTS, W_MEM, BS), lambda i: (0, 0, i)),
        pl.BlockSpec((BS, W_MEM), lambda i: (i, 0)),
        pl.BlockSpec((N_SLOTS, BS), lambda i: (0, i)),
    ]
    return pl.pallas_call(
        _story_body, out_shape=out_shape, grid=(GS,),
        in_specs=in_specs, out_specs=out_specs,
        compiler_params=pltpu.CompilerParams(vmem_limit_bytes=100 * 1024 * 1024),
    )(sx, Wg, b_story, Wif, bif)


GO = 4
BO = 2 * B // GO


def _opt_body(x_ref, Wg_ref, b2_ref, bpl_ref, Wif_ref, bif_ref,
              Wout_ref, bout_ref, Wfc_ref, bfc_ref,
              h_ref, c_ref, M_ref, r_ref, Mn_ref,
              out_ref, wr_ref, ww_ref, a_scr):
    Wg = Wg_ref[...]
    Wif = Wif_ref[...]
    bif = bif_ref[...]
    bias = b2_ref[0]

    def step(t, carry):
        h, c, MT, r, Mn = carry
        x = x_ref[t]
        inp = jnp.concatenate([x, r, h], axis=-1)
        h, c, MT, r, Mn, wrT, wwT = _dnc_step(inp, h, c, MT, r, Mn, Wg, bias, Wif, bif)
        wr_ref[t] = wrT
        ww_ref[t] = wwT
        return (h, c, MT, r, Mn)

    state0 = (h_ref[...], c_ref[...], M_ref[...], r_ref[...], Mn_ref[...])
    h, c, MT, r, Mn = lax.fori_loop(0, T_OPT, step, state0)

    inp = jnp.concatenate([jnp.zeros((BO, XDIM), jnp.float32), r, h], axis=-1)
    h, c, MT, r, Mn, wrT, wwT = _dnc_step(inp, h, c, MT, r, Mn, Wg, bpl_ref[...], Wif, bif)
    wr_ref[T_OPT] = wrT
    ww_ref[T_OPT] = wwT

    a = jnp.dot(jnp.concatenate([h, r], axis=-1), Wout_ref[...],
                preferred_element_type=jnp.float32) + bout_ref[...]
    i = pl.program_id(0)

    @pl.when(i % 2 == 0)
    def _():
        a_scr[...] = a

    @pl.when(i % 2 == 1)
    def _():
        combined = jnp.concatenate([a_scr[...], a], axis=-1)
        out_ref[...] = jnp.dot(combined, Wfc_ref[...],
                               preferred_element_type=jnp.float32) + bfc_ref[...]


def _run_options(ox, Wg, b2, b_plain, Wif, bif, Wout, bout, Wfc, bfc,
                 h, c, M, r, Mn):
    out_shape = [
        jax.ShapeDtypeStruct((B, 2), jnp.float32),
        jax.ShapeDtypeStruct((T_OPT + 1, N_SLOTS, 2 * B), jnp.float32),
        jax.ShapeDtypeStruct((T_OPT + 1, N_SLOTS, 2 * B), jnp.float32),
    ]
    in_specs = [
        pl.BlockSpec((T_OPT, BO, XDIM), lambda i: (0, i, 0)),
        pl.BlockSpec((2 * H, 4 * H), lambda i: (0, 0)),
        pl.BlockSpec((1, 1, 4 * H), lambda i: (i % 2, 0, 0)),
        pl.BlockSpec((1, 4 * H), lambda i: (0, 0)),
        pl.BlockSpec((H, 384), lambda i: (0, 0)),
        pl.BlockSpec((1, 384), lambda i: (0, 0)),
        pl.BlockSpec((H + W_MEM, W_MEM), lambda i: (0, 0)),
        pl.BlockSpec((1, W_MEM), lambda i: (0, 0)),
        pl.BlockSpec((H, 2), lambda i: (0, 0)),
        pl.BlockSpec((1, 2), lambda i: (0, 0)),
        pl.BlockSpec((BO, H), lambda i: (i // 2, 0)),
        pl.BlockSpec((BO, H), lambda i: (i // 2, 0)),
        pl.BlockSpec((N_SLOTS, W_MEM, BO), lambda i: (0, 0, i // 2)),
        pl.BlockSpec((BO, W_MEM), lambda i: (i // 2, 0)),
        pl.BlockSpec((N_SLOTS, BO), lambda i: (0, i // 2)),
    ]
    out_specs = [
        pl.BlockSpec((BO, 2), lambda i: (i // 2, 0)),
        pl.BlockSpec((T_OPT + 1, N_SLOTS, BO), lambda i: (0, 0, i)),
        pl.BlockSpec((T_OPT + 1, N_SLOTS, BO), lambda i: (0, 0, i)),
    ]
    return pl.pallas_call(
        _opt_body, out_shape=out_shape, grid=(GO,),
        in_specs=in_specs, out_specs=out_specs,
        scratch_shapes=[pltpu.VMEM((BO, W_MEM), jnp.float32)],
        compiler_params=pltpu.CompilerParams(vmem_limit_bytes=100 * 1024 * 1024),
    )(ox, Wg, b2, b_plain, Wif, bif, Wout, bout, Wfc, bfc,
      h, c, M, r, Mn)


def kernel(emb, Wx, Wh, b_lstm, W_if, b_if, W_out, b_out, W_fc, b_fc,
           story_batch, option1_batch, option2_batch):
    f32 = jnp.float32
    table = jnp.pad(emb, ((0, 0), (0, TPAD - EMBED_DIM)))
    opt_ids = jnp.concatenate([
        option1_batch[0:BO], option2_batch[0:BO],
        option1_batch[BO:B], option2_batch[BO:B]], axis=0)
    ids = jnp.concatenate([
        story_batch.T.reshape(-1), opt_ids.T.reshape(-1)
    ]).astype(jnp.int32).reshape(NW, NCHUNK, CHUNK)

    g = _sc_gather(table, ids)

    n_s = B * T_STORY
    sx = g[0:n_s, 0:XDIM].reshape(T_STORY, B, XDIM)
    ox = g[n_s:, 0:XDIM].reshape(T_OPT, 2 * B, XDIM)

    Wg = jnp.concatenate([Wx, Wh], axis=0)
    b_story = (b_lstm + Wx[61])[None].astype(f32)
    b_o1 = (b_lstm + Wx[62])[None].astype(f32)
    b_o2 = (b_lstm + Wx[63])[None].astype(f32)
    b_plain = b_lstm[None].astype(f32)
    Wif = jnp.concatenate([
        W_if[:, 0:64], W_if[:, 65:129], W_if[:, 129:193], W_if[:, 193:257],
        W_if[:, 64:65], W_if[:, 257:258], jnp.zeros((H, 126), f32)
    ], axis=1)
    bif = jnp.concatenate([
        b_if[0:64], b_if[65:129], b_if[129:193], b_if[193:257],
        b_if[64:65], b_if[257:258], jnp.zeros((126,), f32)
    ])[None]

    wr_s, ww_s, h, c, M, r, Mn = _run_story(sx, Wg, b_story, Wif, bif)
    b2 = jnp.concatenate([b_o1, b_o2], axis=0)[:, None, :]
    logits, wr_o, ww_o = _run_options(
        ox, Wg, b2, b_plain, Wif, bif,
        W_out, b_out[None], W_fc, b_fc[None], h, c, M, r, Mn)

    hr_s = wr_s.transpose(2, 0, 1)
    hw_s = ww_s.transpose(2, 0, 1)
    wr_o = wr_o.transpose(2, 0, 1)
    ww_o = ww_o.transpose(2, 0, 1)
    hr1 = jnp.concatenate([wr_o[0:BO], wr_o[2 * BO:3 * BO]], axis=0)
    hr2 = jnp.concatenate([wr_o[BO:2 * BO], wr_o[3 * BO:4 * BO]], axis=0)
    hw1 = jnp.concatenate([ww_o[0:BO], ww_o[2 * BO:3 * BO]], axis=0)
    hw2 = jnp.concatenate([ww_o[BO:2 * BO], ww_o[3 * BO:4 * BO]], axis=0)
    return (logits, (hr_s, hr1, hr2), (hw_s, hw1, hw2))


# speedup vs baseline: 8.6468x; 1.0744x over previous
import functools

import jax
import jax.numpy as jnp
from jax import lax
from jax.experimental import pallas as pl
from jax.experimental.pallas import tpu as pltpu
from jax.experimental.pallas import tpu_sc as plsc

EMBED_DIM = 61
XDIM = 64
H = 128
N_SLOTS = 64
W_MEM = 64
CLIP = 20.0
B = 256
T_STORY = 50
T_OPT = 20

NW = 32
TOT_IDS = B * (T_STORY + 2 * T_OPT)
B_PER_W = TOT_IDS // NW
CHUNK = 120
NCHUNK = B_PER_W // CHUNK
TPAD = 128


def _sc_gather_body(table_hbm, idx_hbm, out_hbm, idx_v, rows_v, sem):
    wid = lax.axis_index("s") * 2 + lax.axis_index("c")
    pltpu.sync_copy(idx_hbm.at[wid], idx_v)
    for j in range(NCHUNK):
        pltpu.async_copy(
            table_hbm.at[idx_v.at[j]],
            rows_v.at[pl.ds(j * CHUNK, CHUNK)],
            sem,
        ).wait()
    pltpu.sync_copy(rows_v, out_hbm.at[pl.ds(wid * B_PER_W, B_PER_W)])


def _sc_gather(table, idx3):
    mesh = plsc.VectorSubcoreMesh(core_axis_name="c", subcore_axis_name="s")
    run = functools.partial(
        pl.kernel,
        _sc_gather_body,
        mesh=mesh,
        out_type=jax.ShapeDtypeStruct((TOT_IDS, TPAD), jnp.float32),
        scratch_types=[
            pltpu.VMEM((NCHUNK, CHUNK), jnp.int32),
            pltpu.VMEM((B_PER_W, TPAD), jnp.float32),
            pltpu.SemaphoreType.DMA,
        ],
    )()
    return run(table, idx3)


def _softmax0(x):
    e = jnp.exp(x - jnp.max(x, axis=0, keepdims=True))
    return e / jnp.sum(e, axis=0, keepdims=True)


def _dnc_step(inp, h, c, MT, r, Mn, Wg, bias, Wif, bif):
    gates = jnp.dot(inp, Wg, preferred_element_type=jnp.float32) + bias
    i_g = gates[:, 0:H]
    f_g = gates[:, H:2 * H]
    g_g = gates[:, 2 * H:3 * H]
    o_g = gates[:, 3 * H:4 * H]
    c_new = jax.nn.sigmoid(f_g) * c + jax.nn.sigmoid(i_g) * jnp.tanh(g_g)
    h_new = jax.nn.sigmoid(o_g) * jnp.tanh(c_new)
    h_new = jnp.clip(h_new, -CLIP, CLIP)

    itf = jnp.dot(h_new, Wif, preferred_element_type=jnp.float32) + bif
    kT = jnp.transpose(itf[:, 0:256])
    wkT = kT[0:64]
    erT = jax.nn.sigmoid(kT[64:128])
    wvT = kT[128:192]
    rkT = kT[192:256]
    bT = jnp.transpose(itf[:, 256:258])
    wbT = jnp.logaddexp(bT[0:1], 0.0)
    rbT = jnp.logaddexp(bT[1:2], 0.0)

    eps = jnp.float32(1e-6)
    wknorm = jnp.sqrt(jnp.sum(wkT * wkT, axis=0, keepdims=True)) + eps
    cw = jnp.sum(MT * wkT[None, :, :], axis=1) / ((Mn + eps) * wknorm)
    wwT = _softmax0(cw * wbT)

    MT_new = MT + wwT[:, None, :] * (wvT[None, :, :] - MT * erT[None, :, :])

    Mn2 = jnp.sqrt(jnp.sum(MT_new * MT_new, axis=1))
    rknorm = jnp.sqrt(jnp.sum(rkT * rkT, axis=0, keepdims=True)) + eps
    cr = jnp.sum(MT_new * rkT[None, :, :], axis=1) / ((Mn2 + eps) * rknorm)
    wrT = _softmax0(cr * rbT)
    rT_new = jnp.sum(wrT[:, None, :] * MT_new, axis=0)
    r_new = jnp.transpose(rT_new)
    return h_new, c_new, MT_new, r_new, Mn2, wrT, wwT


GS = 1
BS = B // GS


def _story_body(x_ref, Wg_ref, b_ref, Wif_ref, bif_ref,
                wr_ref, ww_ref, ho_ref, co_ref, Mo_ref, ro_ref, Mno_ref):
    Wg = Wg_ref[...]
    Wif = Wif_ref[...]
    bias = b_ref[...]
    bif = bif_ref[...]

    def step(t, carry):
        h, c, MT, r, Mn = carry
        x = x_ref[t]
        inp = jnp.concatenate([x, r, h], axis=-1)
        h, c, MT, r, Mn, wrT, wwT = _dnc_step(inp, h, c, MT, r, Mn, Wg, bias, Wif, bif)
        wr_ref[t] = wrT
        ww_ref[t] = wwT
        return (h, c, MT, r, Mn)

    init = (jnp.zeros((BS, H), jnp.float32), jnp.zeros((BS, H), jnp.float32),
            jnp.zeros((N_SLOTS, W_MEM, BS), jnp.float32), jnp.zeros((BS, W_MEM), jnp.float32),
            jnp.zeros((N_SLOTS, BS), jnp.float32))
    h, c, MT, r, Mn = lax.fori_loop(0, T_STORY, step, init)
    ho_ref[...] = h
    co_ref[...] = c
    Mo_ref[...] = MT
    ro_ref[...] = r
    Mno_ref[...] = Mn


def _run_story(sx, Wg, b_story, Wif, bif):
    out_shape = [
        jax.ShapeDtypeStruct((T_STORY, N_SLOTS, B), jnp.float32),
        jax.ShapeDtypeStruct((T_STORY, N_SLOTS, B), jnp.float32),
        jax.ShapeDtypeStruct((B, H), jnp.float32),
        jax.ShapeDtypeStruct((B, H), jnp.float32),
        jax.ShapeDtypeStruct((N_SLOTS, W_MEM, B), jnp.float32),
        jax.ShapeDtypeStruct((B, W_MEM), jnp.float32),
        jax.ShapeDtypeStruct((N_SLOTS, B), jnp.float32),
    ]
    in_specs = [
        pl.BlockSpec((T_STORY, BS, XDIM), lambda i: (0, i, 0)),
        pl.BlockSpec((2 * H, 4 * H), lambda i: (0, 0)),
        pl.BlockSpec((1, 4 * H), lambda i: (0, 0)),
        pl.BlockSpec((H, 384), lambda i: (0, 0)),
        pl.BlockSpec((1, 384), lambda i: (0, 0)),
    ]
    out_specs = [
        pl.BlockSpec((T_STORY, N_SLOTS, BS), lambda i: (0, 0, i)),
        pl.BlockSpec((T_STORY, N_SLOTS, BS), lambda i: (0, 0, i)),
        pl.BlockSpec((BS, H), lambda i: (i, 0)),
        pl.BlockSpec((BS, H), lambda i: (i, 0)),
        pl.BlockSpec((N_SLOTS, W_MEM, BS), lambda i: (0, 0, i)),
        pl.BlockSpec((BS, W_MEM), lambda i: (i, 0)),
        pl.BlockSpec((N_SLOTS, BS), lambda i: (0, i)),
    ]
    return pl.pallas_call(
        _story_body, out_shape=out_shape, grid=(GS,),
        in_specs=in_specs, out_specs=out_specs,
        compiler_params=pltpu.CompilerParams(vmem_limit_bytes=100 * 1024 * 1024),
    )(sx, Wg, b_story, Wif, bif)


GO = 2
BO = 2 * B // GO


def _opt_body(x_ref, Wg_ref, b2_ref, bpl_ref, Wif_ref, bif_ref,
              Wout_ref, bout_ref, Wfc_ref, bfc_ref,
              h_ref, c_ref, M_ref, r_ref, Mn_ref,
              out_ref, wr_ref, ww_ref, a_scr):
    Wg = Wg_ref[...]
    Wif = Wif_ref[...]
    bif = bif_ref[...]
    bias = b2_ref[0]

    def step(t, carry):
        h, c, MT, r, Mn = carry
        x = x_ref[t]
        inp = jnp.concatenate([x, r, h], axis=-1)
        h, c, MT, r, Mn, wrT, wwT = _dnc_step(inp, h, c, MT, r, Mn, Wg, bias, Wif, bif)
        wr_ref[t] = wrT
        ww_ref[t] = wwT
        return (h, c, MT, r, Mn)

    state0 = (h_ref[...], c_ref[...], M_ref[...], r_ref[...], Mn_ref[...])
    h, c, MT, r, Mn = lax.fori_loop(0, T_OPT, step, state0)

    inp = jnp.concatenate([jnp.zeros((BO, XDIM), jnp.float32), r, h], axis=-1)
    h, c, MT, r, Mn, wrT, wwT = _dnc_step(inp, h, c, MT, r, Mn, Wg, bpl_ref[...], Wif, bif)
    wr_ref[T_OPT] = wrT
    ww_ref[T_OPT] = wwT

    a = jnp.dot(jnp.concatenate([h, r], axis=-1), Wout_ref[...],
                preferred_element_type=jnp.float32) + bout_ref[...]
    i = pl.program_id(0)

    @pl.when(i % 2 == 0)
    def _():
        a_scr[...] = a

    @pl.when(i % 2 == 1)
    def _():
        combined = jnp.concatenate([a_scr[...], a], axis=-1)
        out_ref[...] = jnp.dot(combined, Wfc_ref[...],
                               preferred_element_type=jnp.float32) + bfc_ref[...]


def _run_options(ox, Wg, b2, b_plain, Wif, bif, Wout, bout, Wfc, bfc,
                 h, c, M, r, Mn):
    out_shape = [
        jax.ShapeDtypeStruct((B, 2), jnp.float32),
        jax.ShapeDtypeStruct((T_OPT + 1, N_SLOTS, 2 * B), jnp.float32),
        jax.ShapeDtypeStruct((T_OPT + 1, N_SLOTS, 2 * B), jnp.float32),
    ]
    in_specs = [
        pl.BlockSpec((T_OPT, BO, XDIM), lambda i: (0, i, 0)),
        pl.BlockSpec((2 * H, 4 * H), lambda i: (0, 0)),
        pl.BlockSpec((1, 1, 4 * H), lambda i: (i % 2, 0, 0)),
        pl.BlockSpec((1, 4 * H), lambda i: (0, 0)),
        pl.BlockSpec((H, 384), lambda i: (0, 0)),
        pl.BlockSpec((1, 384), lambda i: (0, 0)),
        pl.BlockSpec((H + W_MEM, W_MEM), lambda i: (0, 0)),
        pl.BlockSpec((1, W_MEM), lambda i: (0, 0)),
        pl.BlockSpec((H, 2), lambda i: (0, 0)),
        pl.BlockSpec((1, 2), lambda i: (0, 0)),
        pl.BlockSpec((B, H), lambda i: (0, 0)),
        pl.BlockSpec((B, H), lambda i: (0, 0)),
        pl.BlockSpec((N_SLOTS, W_MEM, B), lambda i: (0, 0, 0)),
        pl.BlockSpec((B, W_MEM), lambda i: (0, 0)),
        pl.BlockSpec((N_SLOTS, B), lambda i: (0, 0)),
    ]
    out_specs = [
        pl.BlockSpec((B, 2), lambda i: (0, 0)),
        pl.BlockSpec((T_OPT + 1, N_SLOTS, BO), lambda i: (0, 0, i)),
        pl.BlockSpec((T_OPT + 1, N_SLOTS, BO), lambda i: (0, 0, i)),
    ]
    return pl.pallas_call(
        _opt_body, out_shape=out_shape, grid=(GO,),
        in_specs=in_specs, out_specs=out_specs,
        scratch_shapes=[pltpu.VMEM((BO, W_MEM), jnp.float32)],
        compiler_params=pltpu.CompilerParams(vmem_limit_bytes=100 * 1024 * 1024),
    )(ox, Wg, b2, b_plain, Wif, bif, Wout, bout, Wfc, bfc,
      h, c, M, r, Mn)


def kernel(emb, Wx, Wh, b_lstm, W_if, b_if, W_out, b_out, W_fc, b_fc,
           story_batch, option1_batch, option2_batch):
    f32 = jnp.float32
    table = jnp.pad(emb, ((0, 0), (0, TPAD - EMBED_DIM)))
    opt_ids = jnp.concatenate([option1_batch, option2_batch], axis=0)
    ids = jnp.concatenate([
        story_batch.T.reshape(-1), opt_ids.T.reshape(-1)
    ]).astype(jnp.int32).reshape(NW, NCHUNK, CHUNK)

    g = _sc_gather(table, ids)

    n_s = B * T_STORY
    sx = g[0:n_s, 0:XDIM].reshape(T_STORY, B, XDIM)
    ox = g[n_s:, 0:XDIM].reshape(T_OPT, 2 * B, XDIM)

    Wg = jnp.concatenate([Wx, Wh], axis=0)
    b_story = (b_lstm + Wx[61])[None].astype(f32)
    b_o1 = (b_lstm + Wx[62])[None].astype(f32)
    b_o2 = (b_lstm + Wx[63])[None].astype(f32)
    b_plain = b_lstm[None].astype(f32)
    Wif = jnp.concatenate([
        W_if[:, 0:64], W_if[:, 65:129], W_if[:, 129:193], W_if[:, 193:257],
        W_if[:, 64:65], W_if[:, 257:258], jnp.zeros((H, 126), f32)
    ], axis=1)
    bif = jnp.concatenate([
        b_if[0:64], b_if[65:129], b_if[129:193], b_if[193:257],
        b_if[64:65], b_if[257:258], jnp.zeros((126,), f32)
    ])[None]

    wr_s, ww_s, h, c, M, r, Mn = _run_story(sx, Wg, b_story, Wif, bif)
    b2 = jnp.concatenate([b_o1, b_o2], axis=0)[:, None, :]
    logits, wr_o, ww_o = _run_options(
        ox, Wg, b2, b_plain, Wif, bif,
        W_out, b_out[None], W_fc, b_fc[None], h, c, M, r, Mn)

    hr_s = wr_s.transpose(2, 0, 1)
    hw_s = ww_s.transpose(2, 0, 1)
    wr_o = wr_o.transpose(2, 0, 1)
    ww_o = ww_o.transpose(2, 0, 1)
    hr1, hr2 = wr_o[0:B], wr_o[B:2 * B]
    hw1, hw2 = ww_o[0:B], ww_o[B:2 * B]
    return (logits, (hr_s, hr1, hr2), (hw_s, hw1, hw2))
